# baseline (speedup 1.0000x reference)
import functools
import math

import jax
import jax.numpy as jnp
from jax import lax
from jax.experimental import pallas as pl
from jax.experimental.pallas import tpu as pltpu
from jax.experimental.pallas import tpu_sc as plsc

N = 10000
HC = 128
NG = 50
NGP = 56
NI = 3
CUT = 10.0
K = 32
E = N * K

RB = 200
CB = 128
KW = K + CB
NPAD = 10240
SENT = 1e30
BIGI = 2**31 - 1

NBI = 200
NBE = NBI * K

NW = 32

F32 = jnp.float32
BF16 = jnp.bfloat16
I32 = jnp.int32


def _ssp(x):
    return jnp.maximum(x, 0.0) + jnp.log(1.0 + jnp.exp(-jnp.abs(x))) - math.log(2.0)


def _prelu(x, a):
    return jnp.maximum(x, 0.0) + a * jnp.minimum(x, 0.0)


def _bdot(x, w):
    return jnp.dot(x.astype(BF16), w, preferred_element_type=F32)


def _edges_body(c0s, c1s, pos_r, bat_r, posT_h, batT_h, src_o, vf_o,
                pbuf, bbuf, ad, ai, td, ti, sp, sb):
    r = pl.program_id(0)
    c0 = c0s[r]
    c1 = c1s[r]
    pi = pos_r[...]
    sqi = jnp.sum(pi * pi, axis=1, keepdims=True)
    pib = pi.astype(BF16)
    bi = bat_r[...]
    ig = lax.broadcasted_iota(I32, (RB, 1), 0) + r * RB
    lane32 = lax.broadcasted_iota(I32, (RB, K), 1)

    td[...] = jnp.full((RB, K), SENT, F32)
    ti[...] = jnp.zeros((RB, K), I32)

    trip = c1 - c0 + 1

    def _chunk_copies(ci, slot):
        off = (c0 + ci) * CB
        return (pltpu.make_async_copy(posT_h.at[:, pl.ds(off, CB)],
                                      pbuf.at[slot], sp.at[slot]),
                pltpu.make_async_copy(batT_h.at[:, pl.ds(off, CB)],
                                      bbuf.at[slot], sb.at[slot]))

    for c in _chunk_copies(0, 0):
        c.start()

    def chunk_body(ci, _):
        slot = lax.rem(ci, 2)
        cp, cb_ = _chunk_copies(ci, slot)
        cp.wait()
        cb_.wait()

        @pl.when(ci + 1 < trip)
        def _():
            for c in _chunk_copies(ci + 1, 1 - slot):
                c.start()

        pj = pbuf[slot]
        bj = bbuf[slot]
        off = (c0 + ci) * CB
        sqj = jnp.sum(pj * pj, axis=0, keepdims=True)
        cross = jnp.dot(pib, pj.astype(BF16), preferred_element_type=F32)
        d2s = jnp.maximum(sqi + sqj - 2.0 * cross, 0.0)
        jg = lax.broadcasted_iota(I32, (1, CB), 1) + off
        ok = (bj == bi) & (jg != ig) & (d2s < CUT * CUT)
        ad[:, 0:K] = td[...]
        ai[:, 0:K] = ti[...]
        ad[:, K:KW] = jnp.where(ok, d2s, SENT)
        ai[:, K:KW] = jnp.broadcast_to(jg, (RB, CB))

        def ex(e, _2):
            a = ad[...]
            m = jnp.min(a, axis=1, keepdims=True)
            eqm = a == m
            av = ai[...]
            sel = jnp.min(jnp.where(eqm, av, BIGI), axis=1, keepdims=True)
            ad[...] = jnp.where(eqm & (av == sel), SENT, a)
            td[...] = jnp.where(lane32 == e, m, td[...])
            ti[...] = jnp.where(lane32 == e, sel, ti[...])
            return 0

        lax.fori_loop(0, K, ex, 0)
        return 0

    lax.fori_loop(0, trip, chunk_body, 0)

    okv = td[...] < 1e29
    src_o[...] = jnp.where(okv, ti[...], 0)
    vf_o[...] = okv.astype(F32)


def _build_edges(pos, batch):
    posT = jnp.pad(pos.T, ((0, 0), (0, NPAD - N)))
    batT = jnp.pad(batch[None, :], ((0, 0), (0, NPAD - N)), constant_values=-1)
    b2d = batch[:, None]
    bfirst = batch[::RB]
    blast = batch[RB - 1::RB]
    starts = jnp.searchsorted(batch, bfirst, side="left").astype(I32)
    ends = jnp.searchsorted(batch, blast, side="right").astype(I32)
    c0 = (starts // CB).astype(I32)
    c1 = ((ends - 1) // CB).astype(I32)

    grid_spec = pltpu.PrefetchScalarGridSpec(
        num_scalar_prefetch=2,
        grid=(N // RB,),
        in_specs=[
            pl.BlockSpec((RB, 3), lambda r, *_: (r, 0)),
            pl.BlockSpec((RB, 1), lambda r, *_: (r, 0)),
            pl.BlockSpec(memory_space=pl.ANY),
            pl.BlockSpec(memory_space=pl.ANY),
        ],
        out_specs=[
            pl.BlockSpec((RB, K), lambda r, *_: (r, 0)),
            pl.BlockSpec((RB, K), lambda r, *_: (r, 0)),
        ],
        scratch_shapes=[
            pltpu.VMEM((2, 3, CB), F32),
            pltpu.VMEM((2, 1, CB), I32),
            pltpu.VMEM((RB, KW), F32),
            pltpu.VMEM((RB, KW), I32),
            pltpu.VMEM((RB, K), F32),
            pltpu.VMEM((RB, K), I32),
            pltpu.SemaphoreType.DMA((2,)),
            pltpu.SemaphoreType.DMA((2,)),
        ],
    )
    return pl.pallas_call(
        _edges_body,
        grid_spec=grid_spec,
        out_shape=[
            jax.ShapeDtypeStruct((N, K), I32),
            jax.ShapeDtypeStruct((N, K), F32),
        ],
    )(c0, c1, pos, b2d, posT, batT)


def _geom_body(ps_r, pd_r, vf_r, d_o, c_o):
    ps3 = ps_r[...].reshape(NBI, K, HC)
    pd = pd_r[...]
    df0 = ps3[:, :, 0] - pd[:, 0:1]
    df1 = ps3[:, :, 1] - pd[:, 1:2]
    df2 = ps3[:, :, 2] - pd[:, 2:3]
    d2 = (df0 * df0 + df1 * df1) + df2 * df2
    ok = vf_r[...] != 0.0
    dd = jnp.where(ok, jnp.sqrt(d2), CUT)
    d_o[...] = dd
    c_o[...] = 0.5 * (jnp.cos(dd * math.pi / CUT) + 1.0) * vf_r[...]


def _geom(ps, pos16, vf):
    return pl.pallas_call(
        _geom_body,
        grid=(N // NBI,),
        in_specs=[
            pl.BlockSpec((NBE, HC), lambda i: (i, 0)),
            pl.BlockSpec((NBI, HC), lambda i: (i, 0)),
            pl.BlockSpec((NBI, K), lambda i: (i, 0)),
        ],
        out_specs=[pl.BlockSpec((NBI, K), lambda i: (i, 0)),
                   pl.BlockSpec((NBI, K), lambda i: (i, 0))],
        out_shape=[jax.ShapeDtypeStruct((N, K), F32),
                   jax.ShapeDtypeStruct((N, K), F32)],
    )(ps, pos16, vf)


def _table_body(embp, l1wb, out):
    e = embp[...]
    out[:, 0:HC] = e
    out[:, HC:2 * HC] = _bdot(e, l1wb[...])


def _table(emb, l1w0):
    embp = jnp.pad(emb, ((0, 104 - emb.shape[0]), (0, 0)))
    return pl.pallas_call(
        _table_body,
        out_shape=jax.ShapeDtypeStruct((104, 2 * HC), F32),
    )(embp, l1w0.astype(BF16))


def _sc_gather(table, idx3, ch, nch, d, bk):
    per_w = ch * nch
    b = per_w * NW
    nb = nch // bk
    assert nb * bk == nch and nb % 2 == 1
    mesh = plsc.VectorSubcoreMesh(core_axis_name="c", subcore_axis_name="s")

    scratch = [
        pltpu.VMEM((nch, ch), I32),
        pltpu.VMEM((bk * ch, d), F32),
        pltpu.SemaphoreType.DMA,
        pltpu.SemaphoreType.DMA,
    ]
    if nb > 1:
        scratch += [
            pltpu.VMEM((bk * ch, d), F32),
            pltpu.SemaphoreType.DMA,
            pltpu.SemaphoreType.DMA,
        ]

    @functools.partial(
        pl.kernel,
        out_type=jax.ShapeDtypeStruct((b, d), F32),
        mesh=mesh,
        scratch_types=scratch,
    )
    def k(tab_h, idx_h, out_h, idx_v, rows0, sg0, so0, *rest):
        wid = lax.axis_index("s") * 2 + lax.axis_index("c")
        base = wid * per_w
        pltpu.sync_copy(idx_h.at[wid], idx_v)

        def fire_g(p, rows, sg):
            for i in range(bk):
                pltpu.async_copy(tab_h.at[idx_v.at[p * bk + i]],
                                 rows.at[pl.ds(i * ch, ch)], sg)

        def wait_g(rows, sg):
            for i in range(bk):
                pltpu.make_async_copy(tab_h.at[pl.ds(0, ch)],
                                      rows.at[pl.ds(i * ch, ch)], sg).wait()

        def fire_o(p, rows, so):
            pltpu.async_copy(rows, out_h.at[pl.ds(base + p * bk * ch, bk * ch)], so)

        def wait_o(rows, so):
            pltpu.make_async_copy(rows, out_h.at[pl.ds(base, bk * ch)], so).wait()

        fire_g(0, rows0, sg0)
        if nb > 1:
            rows1, sg1, so1 = rest

            def pair(pr, _):
                p0 = 2 * pr
                fire_g(p0 + 1, rows1, sg1)
                wait_g(rows0, sg0)
                fire_o(p0, rows0, so0)
                wait_g(rows1, sg1)
                fire_o(p0 + 1, rows1, so1)
                wait_o(rows0, so0)
                fire_g(p0 + 2, rows0, sg0)
                wait_o(rows1, so1)
                return 0

            lax.fori_loop(0, (nb - 1) // 2, pair, 0)
        wait_g(rows0, sg0)
        fire_o(nb - 1, rows0, so0)
        wait_o(rows0, so0)

    return k(table, idx3)


def _inter_body(is_last, xj_r, de_r, ce_r, h_r, offs, cf, w1, b1, w2, b2,
                l2w, l2b, lw, lb, *rest):
    dblk = de_r[...]
    cblk = ce_r[...]
    off = offs[...]
    cfv = cf[0, 0]
    xj3 = xj_r[...]
    ea = jnp.concatenate(
        [jnp.exp(cfv * (dblk[:, j:j + 1] - off) ** 2) for j in range(K)],
        axis=0)
    u = _ssp(_bdot(ea, w1[...]) + b1[...])
    W3 = (_bdot(u, w2[...]) + b2[...]).reshape(K, NBI, HC)
    agg = None
    for j in range(K):
        pj = xj3[j] * (W3[j] * cblk[:, j:j + 1])
        agg = pj if agg is None else agg + pj
    o = _ssp(_bdot(agg, l2w[...]) + l2b[...])
    o = _bdot(o, lw[...]) + lb[...]
    hn = h_r[...] + o
    if not is_last:
        l1n, h_o, xh_o = rest
        h_o[...] = hn
        xh_o[...] = _bdot(hn, l1n[...])
    else:
        pw1, pb1, pa1, pw2, pb2, pa2, pw3, pb3, y_o = rest
        y1 = _prelu(_bdot(hn, pw1[...]) + pb1[...], pa1[0, 0])
        y2 = _prelu(_bdot(y1, pw2[...]) + pb2[...], pa2[0, 0])
        y_o[...] = _bdot(y2, pw3[...]) + pb3[0, 0]


def _full(shape):
    return pl.BlockSpec(shape, lambda i: tuple(0 for _ in shape))


def _interact(xj, d_e, c_e, h, offs, cf, w1p, b1, w2, b2, l2w, l2b, lw, lb,
              extra, is_last):
    in_specs = [
        pl.BlockSpec((K, NBI, HC), lambda i: (0, i, 0)),
        pl.BlockSpec((NBI, K), lambda i: (i, 0)),
        pl.BlockSpec((NBI, K), lambda i: (i, 0)),
        pl.BlockSpec((NBI, HC), lambda i: (i, 0)),
        _full((1, NGP)), _full((1, 1)),
        _full((NGP, HC)), _full((1, HC)), _full((HC, HC)), _full((1, HC)),
        _full((HC, HC)), _full((1, HC)), _full((HC, HC)), _full((1, HC)),
    ]
    if not is_last:
        in_specs += [_full((HC, HC))]
        out_specs = [pl.BlockSpec((NBI, HC), lambda i: (i, 0)),
                     pl.BlockSpec((NBI, HC), lambda i: (i, 0))]
        out_shape = [jax.ShapeDtypeStruct((N, HC), F32),
                     jax.ShapeDtypeStruct((N, HC), F32)]
    else:
        in_specs += [_full((HC, HC // 2)), _full((1, HC // 2)), _full((1, 1)),
                     _full((HC // 2, HC // 4)), _full((1, HC // 4)),
                     _full((1, 1)), _full((HC // 4, 1)), _full((1, 1))]
        out_specs = [pl.BlockSpec((NBI, 1), lambda i: (i, 0))]
        out_shape = [jax.ShapeDtypeStruct((N, 1), F32)]
    return pl.pallas_call(
        functools.partial(_inter_body, is_last),
        grid=(N // NBI,),
        in_specs=in_specs,
        out_specs=out_specs,
        out_shape=out_shape,
    )(xj, d_e, c_e, h, offs, cf, w1p, b1, w2, b2, l2w, l2b, lw, lb, *extra)


def kernel(z, pos, batch, emb, mlp_w1, mlp_b1, mlp_w2, mlp_b2, lin1_w,
           lin2_w, lin2_b, lin_w, lin_b, p_w1, p_b1, p_a1, p_w2, p_b2,
           p_a2, p_w3, p_b3):
    pos = pos.astype(F32)
    batch = batch.astype(I32)
    z = z.astype(I32)

    src, vf = _build_edges(pos, batch)
    src3n = src.reshape(NW, 125, 80)
    src3j = src.T.reshape(NW, 125, 80)

    pos_p = jnp.pad(pos, ((0, 0), (0, HC - 3)))
    ps = _sc_gather(pos_p, src3n, 80, 125, HC, 5)
    d_nm, c_nm = _geom(ps, pos_p, vf)

    offs_lin = jnp.linspace(0.0, CUT, NG).astype(F32)
    offs = jnp.pad(offs_lin, (0, NGP - NG), constant_values=1e6)[None]
    cf = (-0.5 / (offs_lin[1] - offs_lin[0]) ** 2).reshape(1, 1)

    table = _table(emb, lin1_w[0])
    zp = jnp.pad(z, (0, NPAD - N)).reshape(NW, 5, 64)
    hx = _sc_gather(table, zp, 64, 5, 2 * HC, 5)
    h = hx[:N, :HC]
    xh = hx[:N, HC:]

    y = None
    for t in range(NI):
        xj = _sc_gather(xh, src3j, 80, 125, HC, 5).reshape(K, N, HC)
        w1p = jnp.pad(mlp_w1[t], ((0, NGP - NG), (0, 0))).astype(BF16)
        args = (xj, d_nm, c_nm, h, offs, cf, w1p, mlp_b1[t][None],
                mlp_w2[t].astype(BF16), mlp_b2[t][None],
                lin2_w[t].astype(BF16), lin2_b[t][None],
                lin_w[t].astype(BF16), lin_b[t][None])
        if t < NI - 1:
            extra = (lin1_w[t + 1].astype(BF16),)
            h, xh = _interact(*args, extra, is_last=False)
        else:
            extra = (p_w1.astype(BF16), p_b1[None], p_a1[None],
                     p_w2.astype(BF16), p_b2[None], p_a2[None],
                     p_w3.astype(BF16), p_b3[None])
            (y,) = _interact(*args, extra, is_last=True)
    return y


# speedup vs baseline: 1.2152x; 1.2152x over previous
import functools
import math

import jax
import jax.numpy as jnp
from jax import lax
from jax.experimental import pallas as pl
from jax.experimental.pallas import tpu as pltpu
from jax.experimental.pallas import tpu_sc as plsc

N = 10000
HC = 128
NG = 50
NGP = 56
NI = 3
CUT = 10.0
K = 32
E = N * K

RB = 200
CB = 512
KW = K + CB
NPAD = 10240
SENT = 1e30
BIGI = 2**31 - 1

NBI = 200
NBE = NBI * K

NW = 32

F32 = jnp.float32
BF16 = jnp.bfloat16
I32 = jnp.int32


def _ssp(x):
    return jnp.maximum(x, 0.0) + jnp.log(1.0 + jnp.exp(-jnp.abs(x))) - math.log(2.0)


def _prelu(x, a):
    return jnp.maximum(x, 0.0) + a * jnp.minimum(x, 0.0)


def _bdot(x, w):
    return jnp.dot(x.astype(BF16), w, preferred_element_type=F32)


def _edges_body(c0s, c1s, pos_r, bat_r, posT_h, batT_h, src_o, vf_o,
                pbuf, bbuf, ad, ai, td, ti, sp, sb):
    r = pl.program_id(0)
    c0 = c0s[r]
    c1 = c1s[r]
    pi = pos_r[...]
    sqi = jnp.sum(pi * pi, axis=1, keepdims=True)
    pib = pi.astype(BF16)
    bi = bat_r[...]
    ig = lax.broadcasted_iota(I32, (RB, 1), 0) + r * RB
    lane32 = lax.broadcasted_iota(I32, (RB, K), 1)

    td[...] = jnp.full((RB, K), SENT, F32)
    ti[...] = jnp.zeros((RB, K), I32)

    trip = c1 - c0 + 1

    def _chunk_copies(ci, slot):
        off = (c0 + ci) * CB
        return (pltpu.make_async_copy(posT_h.at[:, pl.ds(off, CB)],
                                      pbuf.at[slot], sp.at[slot]),
                pltpu.make_async_copy(batT_h.at[:, pl.ds(off, CB)],
                                      bbuf.at[slot], sb.at[slot]))

    for c in _chunk_copies(0, 0):
        c.start()

    def chunk_body(ci, _):
        slot = lax.rem(ci, 2)
        cp, cb_ = _chunk_copies(ci, slot)
        cp.wait()
        cb_.wait()

        @pl.when(ci + 1 < trip)
        def _():
            for c in _chunk_copies(ci + 1, 1 - slot):
                c.start()

        pj = pbuf[slot]
        bj = bbuf[slot]
        off = (c0 + ci) * CB
        sqj = jnp.sum(pj * pj, axis=0, keepdims=True)
        cross = jnp.dot(pib, pj.astype(BF16), preferred_element_type=F32)
        d2s = jnp.maximum(sqi + sqj - 2.0 * cross, 0.0)
        jg = lax.broadcasted_iota(I32, (1, CB), 1) + off
        ok = (bj == bi) & (jg != ig) & (d2s < CUT * CUT)
        ad[:, 0:K] = td[...]
        ai[:, 0:K] = ti[...]
        ad[:, K:KW] = jnp.where(ok, d2s, SENT)
        ai[:, K:KW] = jnp.broadcast_to(jg, (RB, CB))

        def ex(e, _2):
            a = ad[...]
            m = jnp.min(a, axis=1, keepdims=True)
            eqm = a == m
            av = ai[...]
            sel = jnp.min(jnp.where(eqm, av, BIGI), axis=1, keepdims=True)
            ad[...] = jnp.where(eqm & (av == sel), SENT, a)
            td[...] = jnp.where(lane32 == e, m, td[...])
            ti[...] = jnp.where(lane32 == e, sel, ti[...])
            return 0

        lax.fori_loop(0, K, ex, 0)
        return 0

    lax.fori_loop(0, trip, chunk_body, 0)

    okv = td[...] < 1e29
    src_o[...] = jnp.where(okv, ti[...], 0)
    vf_o[...] = okv.astype(F32)


def _build_edges(pos, batch):
    posT = jnp.pad(pos.T, ((0, 0), (0, NPAD - N)))
    batT = jnp.pad(batch[None, :], ((0, 0), (0, NPAD - N)), constant_values=-1)
    b2d = batch[:, None]
    bfirst = batch[::RB]
    blast = batch[RB - 1::RB]
    starts = jnp.searchsorted(batch, bfirst, side="left").astype(I32)
    ends = jnp.searchsorted(batch, blast, side="right").astype(I32)
    c0 = (starts // CB).astype(I32)
    c1 = ((ends - 1) // CB).astype(I32)

    grid_spec = pltpu.PrefetchScalarGridSpec(
        num_scalar_prefetch=2,
        grid=(N // RB,),
        in_specs=[
            pl.BlockSpec((RB, 3), lambda r, *_: (r, 0)),
            pl.BlockSpec((RB, 1), lambda r, *_: (r, 0)),
            pl.BlockSpec(memory_space=pl.ANY),
            pl.BlockSpec(memory_space=pl.ANY),
        ],
        out_specs=[
            pl.BlockSpec((RB, K), lambda r, *_: (r, 0)),
            pl.BlockSpec((RB, K), lambda r, *_: (r, 0)),
        ],
        scratch_shapes=[
            pltpu.VMEM((2, 3, CB), F32),
            pltpu.VMEM((2, 1, CB), I32),
            pltpu.VMEM((RB, KW), F32),
            pltpu.VMEM((RB, KW), I32),
            pltpu.VMEM((RB, K), F32),
            pltpu.VMEM((RB, K), I32),
            pltpu.SemaphoreType.DMA((2,)),
            pltpu.SemaphoreType.DMA((2,)),
        ],
    )
    return pl.pallas_call(
        _edges_body,
        grid_spec=grid_spec,
        out_shape=[
            jax.ShapeDtypeStruct((N, K), I32),
            jax.ShapeDtypeStruct((N, K), F32),
        ],
    )(c0, c1, pos, b2d, posT, batT)


def _geom_body(ps_r, pd_r, vf_r, d_o, c_o):
    ps3 = ps_r[...].reshape(NBI, K, HC)
    pd = pd_r[...]
    df0 = ps3[:, :, 0] - pd[:, 0:1]
    df1 = ps3[:, :, 1] - pd[:, 1:2]
    df2 = ps3[:, :, 2] - pd[:, 2:3]
    d2 = (df0 * df0 + df1 * df1) + df2 * df2
    ok = vf_r[...] != 0.0
    dd = jnp.where(ok, jnp.sqrt(d2), CUT)
    d_o[...] = dd
    c_o[...] = 0.5 * (jnp.cos(dd * math.pi / CUT) + 1.0) * vf_r[...]


def _geom(ps, pos16, vf):
    return pl.pallas_call(
        _geom_body,
        grid=(N // NBI,),
        in_specs=[
            pl.BlockSpec((NBE, HC), lambda i: (i, 0)),
            pl.BlockSpec((NBI, HC), lambda i: (i, 0)),
            pl.BlockSpec((NBI, K), lambda i: (i, 0)),
        ],
        out_specs=[pl.BlockSpec((NBI, K), lambda i: (i, 0)),
                   pl.BlockSpec((NBI, K), lambda i: (i, 0))],
        out_shape=[jax.ShapeDtypeStruct((N, K), F32),
                   jax.ShapeDtypeStruct((N, K), F32)],
    )(ps, pos16, vf)


def _table_body(embp, l1wb, out):
    e = embp[...]
    out[:, 0:HC] = e
    out[:, HC:2 * HC] = _bdot(e, l1wb[...])


def _table(emb, l1w0):
    embp = jnp.pad(emb, ((0, 104 - emb.shape[0]), (0, 0)))
    return pl.pallas_call(
        _table_body,
        out_shape=jax.ShapeDtypeStruct((104, 2 * HC), F32),
    )(embp, l1w0.astype(BF16))


def _sc_gather(table, idx3, ch, nch, d, bk):
    per_w = ch * nch
    b = per_w * NW
    nb = nch // bk
    assert nb * bk == nch and nb % 2 == 1
    mesh = plsc.VectorSubcoreMesh(core_axis_name="c", subcore_axis_name="s")

    scratch = [
        pltpu.VMEM((nch, ch), I32),
        pltpu.VMEM((bk * ch, d), F32),
        pltpu.SemaphoreType.DMA,
        pltpu.SemaphoreType.DMA,
    ]
    if nb > 1:
        scratch += [
            pltpu.VMEM((bk * ch, d), F32),
            pltpu.SemaphoreType.DMA,
            pltpu.SemaphoreType.DMA,
        ]

    @functools.partial(
        pl.kernel,
        out_type=jax.ShapeDtypeStruct((b, d), F32),
        mesh=mesh,
        scratch_types=scratch,
    )
    def k(tab_h, idx_h, out_h, idx_v, rows0, sg0, so0, *rest):
        wid = lax.axis_index("s") * 2 + lax.axis_index("c")
        base = wid * per_w
        pltpu.sync_copy(idx_h.at[wid], idx_v)

        def fire_g(p, rows, sg):
            for i in range(bk):
                pltpu.async_copy(tab_h.at[idx_v.at[p * bk + i]],
                                 rows.at[pl.ds(i * ch, ch)], sg)

        def wait_g(rows, sg):
            for i in range(bk):
                pltpu.make_async_copy(tab_h.at[pl.ds(0, ch)],
                                      rows.at[pl.ds(i * ch, ch)], sg).wait()

        def fire_o(p, rows, so):
            pltpu.async_copy(rows, out_h.at[pl.ds(base + p * bk * ch, bk * ch)], so)

        def wait_o(rows, so):
            pltpu.make_async_copy(rows, out_h.at[pl.ds(base, bk * ch)], so).wait()

        fire_g(0, rows0, sg0)
        if nb > 1:
            rows1, sg1, so1 = rest

            def pair(pr, _):
                p0 = 2 * pr
                fire_g(p0 + 1, rows1, sg1)
                wait_g(rows0, sg0)
                fire_o(p0, rows0, so0)
                wait_g(rows1, sg1)
                fire_o(p0 + 1, rows1, so1)
                wait_o(rows0, so0)
                fire_g(p0 + 2, rows0, sg0)
                wait_o(rows1, so1)
                return 0

            lax.fori_loop(0, (nb - 1) // 2, pair, 0)
        wait_g(rows0, sg0)
        fire_o(nb - 1, rows0, so0)
        wait_o(rows0, so0)

    return k(table, idx3)


def _inter_body(is_last, xj_r, de_r, ce_r, h_r, offs, cf, w1, b1, w2, b2,
                l2w, l2b, lw, lb, *rest):
    dblk = de_r[...]
    cblk = ce_r[...]
    off = offs[...]
    cfv = cf[0, 0]
    xj3 = xj_r[...]
    ea = jnp.concatenate(
        [jnp.exp(cfv * (dblk[:, j:j + 1] - off) ** 2) for j in range(K)],
        axis=0)
    u = _ssp(_bdot(ea, w1[...]) + b1[...])
    W3 = (_bdot(u, w2[...]) + b2[...]).reshape(K, NBI, HC)
    agg = None
    for j in range(K):
        pj = xj3[j] * (W3[j] * cblk[:, j:j + 1])
        agg = pj if agg is None else agg + pj
    o = _ssp(_bdot(agg, l2w[...]) + l2b[...])
    o = _bdot(o, lw[...]) + lb[...]
    hn = h_r[...] + o
    if not is_last:
        l1n, h_o, xh_o = rest
        h_o[...] = hn
        xh_o[...] = _bdot(hn, l1n[...])
    else:
        pw1, pb1, pa1, pw2, pb2, pa2, pw3, pb3, y_o = rest
        y1 = _prelu(_bdot(hn, pw1[...]) + pb1[...], pa1[0, 0])
        y2 = _prelu(_bdot(y1, pw2[...]) + pb2[...], pa2[0, 0])
        y_o[...] = _bdot(y2, pw3[...]) + pb3[0, 0]


def _full(shape):
    return pl.BlockSpec(shape, lambda i: tuple(0 for _ in shape))


def _interact(xj, d_e, c_e, h, offs, cf, w1p, b1, w2, b2, l2w, l2b, lw, lb,
              extra, is_last):
    in_specs = [
        pl.BlockSpec((K, NBI, HC), lambda i: (0, i, 0)),
        pl.BlockSpec((NBI, K), lambda i: (i, 0)),
        pl.BlockSpec((NBI, K), lambda i: (i, 0)),
        pl.BlockSpec((NBI, HC), lambda i: (i, 0)),
        _full((1, NGP)), _full((1, 1)),
        _full((NGP, HC)), _full((1, HC)), _full((HC, HC)), _full((1, HC)),
        _full((HC, HC)), _full((1, HC)), _full((HC, HC)), _full((1, HC)),
    ]
    if not is_last:
        in_specs += [_full((HC, HC))]
        out_specs = [pl.BlockSpec((NBI, HC), lambda i: (i, 0)),
                     pl.BlockSpec((NBI, HC), lambda i: (i, 0))]
        out_shape = [jax.ShapeDtypeStruct((N, HC), F32),
                     jax.ShapeDtypeStruct((N, HC), F32)]
    else:
        in_specs += [_full((HC, HC // 2)), _full((1, HC // 2)), _full((1, 1)),
                     _full((HC // 2, HC // 4)), _full((1, HC // 4)),
                     _full((1, 1)), _full((HC // 4, 1)), _full((1, 1))]
        out_specs = [pl.BlockSpec((NBI, 1), lambda i: (i, 0))]
        out_shape = [jax.ShapeDtypeStruct((N, 1), F32)]
    return pl.pallas_call(
        functools.partial(_inter_body, is_last),
        grid=(N // NBI,),
        in_specs=in_specs,
        out_specs=out_specs,
        out_shape=out_shape,
    )(xj, d_e, c_e, h, offs, cf, w1p, b1, w2, b2, l2w, l2b, lw, lb, *extra)


def kernel(z, pos, batch, emb, mlp_w1, mlp_b1, mlp_w2, mlp_b2, lin1_w,
           lin2_w, lin2_b, lin_w, lin_b, p_w1, p_b1, p_a1, p_w2, p_b2,
           p_a2, p_w3, p_b3):
    pos = pos.astype(F32)
    batch = batch.astype(I32)
    z = z.astype(I32)

    src, vf = _build_edges(pos, batch)
    src3n = src.reshape(NW, 125, 80)
    src3j = src.T.reshape(NW, 125, 80)

    pos_p = jnp.pad(pos, ((0, 0), (0, HC - 3)))
    ps = _sc_gather(pos_p, src3n, 80, 125, HC, 5)
    d_nm, c_nm = _geom(ps, pos_p, vf)

    offs_lin = jnp.linspace(0.0, CUT, NG).astype(F32)
    offs = jnp.pad(offs_lin, (0, NGP - NG), constant_values=1e6)[None]
    cf = (-0.5 / (offs_lin[1] - offs_lin[0]) ** 2).reshape(1, 1)

    table = _table(emb, lin1_w[0])
    zp = jnp.pad(z, (0, NPAD - N)).reshape(NW, 5, 64)
    hx = _sc_gather(table, zp, 64, 5, 2 * HC, 5)
    h = hx[:N, :HC]
    xh = hx[:N, HC:]

    y = None
    for t in range(NI):
        xj = _sc_gather(xh, src3j, 80, 125, HC, 5).reshape(K, N, HC)
        w1p = jnp.pad(mlp_w1[t], ((0, NGP - NG), (0, 0))).astype(BF16)
        args = (xj, d_nm, c_nm, h, offs, cf, w1p, mlp_b1[t][None],
                mlp_w2[t].astype(BF16), mlp_b2[t][None],
                lin2_w[t].astype(BF16), lin2_b[t][None],
                lin_w[t].astype(BF16), lin_b[t][None])
        if t < NI - 1:
            extra = (lin1_w[t + 1].astype(BF16),)
            h, xh = _interact(*args, extra, is_last=False)
        else:
            extra = (p_w1.astype(BF16), p_b1[None], p_a1[None],
                     p_w2.astype(BF16), p_b2[None], p_a2[None],
                     p_w3.astype(BF16), p_b3[None])
            (y,) = _interact(*args, extra, is_last=True)
    return y


# speedup vs baseline: 1.2680x; 1.0435x over previous
import functools
import math

import jax
import jax.numpy as jnp
from jax import lax
from jax.experimental import pallas as pl
from jax.experimental.pallas import tpu as pltpu
from jax.experimental.pallas import tpu_sc as plsc

N = 10000
HC = 128
NG = 50
NGP = 56
NI = 3
CUT = 10.0
K = 32
E = N * K

RB = 400
CB = 512
KW = K + CB
NPAD = 10240
SENT = 1e30
BIGI = 2**31 - 1

NBI = 200
NBE = NBI * K

NW = 32

F32 = jnp.float32
BF16 = jnp.bfloat16
I32 = jnp.int32


def _ssp(x):
    return jnp.maximum(x, 0.0) + jnp.log(1.0 + jnp.exp(-jnp.abs(x))) - math.log(2.0)


def _prelu(x, a):
    return jnp.maximum(x, 0.0) + a * jnp.minimum(x, 0.0)


def _bdot(x, w):
    return jnp.dot(x.astype(BF16), w, preferred_element_type=F32)


def _edges_body(c0s, c1s, pos_r, bat_r, posT_h, batT_h, src_o, vf_o,
                pbuf, bbuf, ad, ai, td, ti, sp, sb):
    r = pl.program_id(0)
    c0 = c0s[r]
    c1 = c1s[r]
    pi = pos_r[...]
    sqi = jnp.sum(pi * pi, axis=1, keepdims=True)
    pib = pi.astype(BF16)
    bi = bat_r[...]
    ig = lax.broadcasted_iota(I32, (RB, 1), 0) + r * RB
    lane32 = lax.broadcasted_iota(I32, (RB, K), 1)

    td[...] = jnp.full((RB, K), SENT, F32)
    ti[...] = jnp.zeros((RB, K), I32)

    trip = c1 - c0 + 1

    def _chunk_copies(ci, slot):
        off = (c0 + ci) * CB
        return (pltpu.make_async_copy(posT_h.at[:, pl.ds(off, CB)],
                                      pbuf.at[slot], sp.at[slot]),
                pltpu.make_async_copy(batT_h.at[:, pl.ds(off, CB)],
                                      bbuf.at[slot], sb.at[slot]))

    for c in _chunk_copies(0, 0):
        c.start()

    def chunk_body(ci, _):
        slot = lax.rem(ci, 2)
        cp, cb_ = _chunk_copies(ci, slot)
        cp.wait()
        cb_.wait()

        @pl.when(ci + 1 < trip)
        def _():
            for c in _chunk_copies(ci + 1, 1 - slot):
                c.start()

        pj = pbuf[slot]
        bj = bbuf[slot]
        off = (c0 + ci) * CB
        sqj = jnp.sum(pj * pj, axis=0, keepdims=True)
        cross = jnp.dot(pib, pj.astype(BF16), preferred_element_type=F32)
        d2s = jnp.maximum(sqi + sqj - 2.0 * cross, 0.0)
        jg = lax.broadcasted_iota(I32, (1, CB), 1) + off
        ok = (bj == bi) & (jg != ig) & (d2s < CUT * CUT)
        ad[:, 0:K] = td[...]
        ai[:, 0:K] = ti[...]
        ad[:, K:KW] = jnp.where(ok, d2s, SENT)
        ai[:, K:KW] = jnp.broadcast_to(jg, (RB, CB))

        def ex(e, _2):
            a = ad[...]
            m = jnp.min(a, axis=1, keepdims=True)
            eqm = a == m
            av = ai[...]
            sel = jnp.min(jnp.where(eqm, av, BIGI), axis=1, keepdims=True)
            ad[...] = jnp.where(eqm & (av == sel), SENT, a)
            td[...] = jnp.where(lane32 == e, m, td[...])
            ti[...] = jnp.where(lane32 == e, sel, ti[...])
            return 0

        lax.fori_loop(0, K, ex, 0)
        return 0

    lax.fori_loop(0, trip, chunk_body, 0)

    okv = td[...] < 1e29
    src_o[...] = jnp.where(okv, ti[...], 0)
    vf_o[...] = okv.astype(F32)


def _build_edges(pos, batch):
    posT = jnp.pad(pos.T, ((0, 0), (0, NPAD - N)))
    batT = jnp.pad(batch[None, :], ((0, 0), (0, NPAD - N)), constant_values=-1)
    b2d = batch[:, None]
    bfirst = batch[::RB]
    blast = batch[RB - 1::RB]
    starts = jnp.searchsorted(batch, bfirst, side="left").astype(I32)
    ends = jnp.searchsorted(batch, blast, side="right").astype(I32)
    c0 = (starts // CB).astype(I32)
    c1 = ((ends - 1) // CB).astype(I32)

    grid_spec = pltpu.PrefetchScalarGridSpec(
        num_scalar_prefetch=2,
        grid=(N // RB,),
        in_specs=[
            pl.BlockSpec((RB, 3), lambda r, *_: (r, 0)),
            pl.BlockSpec((RB, 1), lambda r, *_: (r, 0)),
            pl.BlockSpec(memory_space=pl.ANY),
            pl.BlockSpec(memory_space=pl.ANY),
        ],
        out_specs=[
            pl.BlockSpec((RB, K), lambda r, *_: (r, 0)),
            pl.BlockSpec((RB, K), lambda r, *_: (r, 0)),
        ],
        scratch_shapes=[
            pltpu.VMEM((2, 3, CB), F32),
            pltpu.VMEM((2, 1, CB), I32),
            pltpu.VMEM((RB, KW), F32),
            pltpu.VMEM((RB, KW), I32),
            pltpu.VMEM((RB, K), F32),
            pltpu.VMEM((RB, K), I32),
            pltpu.SemaphoreType.DMA((2,)),
            pltpu.SemaphoreType.DMA((2,)),
        ],
    )
    return pl.pallas_call(
        _edges_body,
        grid_spec=grid_spec,
        out_shape=[
            jax.ShapeDtypeStruct((N, K), I32),
            jax.ShapeDtypeStruct((N, K), F32),
        ],
    )(c0, c1, pos, b2d, posT, batT)


def _geom_body(ps_r, pd_r, vf_r, d_o, c_o):
    ps3 = ps_r[...].reshape(NBI, K, HC)
    pd = pd_r[...]
    df0 = ps3[:, :, 0] - pd[:, 0:1]
    df1 = ps3[:, :, 1] - pd[:, 1:2]
    df2 = ps3[:, :, 2] - pd[:, 2:3]
    d2 = (df0 * df0 + df1 * df1) + df2 * df2
    ok = vf_r[...] != 0.0
    dd = jnp.where(ok, jnp.sqrt(d2), CUT)
    d_o[...] = dd
    c_o[...] = 0.5 * (jnp.cos(dd * math.pi / CUT) + 1.0) * vf_r[...]


def _geom(ps, pos16, vf):
    return pl.pallas_call(
        _geom_body,
        grid=(N // NBI,),
        in_specs=[
            pl.BlockSpec((NBE, HC), lambda i: (i, 0)),
            pl.BlockSpec((NBI, HC), lambda i: (i, 0)),
            pl.BlockSpec((NBI, K), lambda i: (i, 0)),
        ],
        out_specs=[pl.BlockSpec((NBI, K), lambda i: (i, 0)),
                   pl.BlockSpec((NBI, K), lambda i: (i, 0))],
        out_shape=[jax.ShapeDtypeStruct((N, K), F32),
                   jax.ShapeDtypeStruct((N, K), F32)],
    )(ps, pos16, vf)


def _table_body(embp, l1wb, out):
    e = embp[...]
    out[:, 0:HC] = e
    out[:, HC:2 * HC] = _bdot(e, l1wb[...])


def _table(emb, l1w0):
    embp = jnp.pad(emb, ((0, 104 - emb.shape[0]), (0, 0)))
    return pl.pallas_call(
        _table_body,
        out_shape=jax.ShapeDtypeStruct((104, 2 * HC), F32),
    )(embp, l1w0.astype(BF16))


def _sc_gather(table, idx3, ch, nch, d, bk):
    per_w = ch * nch
    b = per_w * NW
    nb = nch // bk
    assert nb * bk == nch and nb % 2 == 1
    mesh = plsc.VectorSubcoreMesh(core_axis_name="c", subcore_axis_name="s")

    scratch = [
        pltpu.VMEM((nch, ch), I32),
        pltpu.VMEM((bk * ch, d), F32),
        pltpu.SemaphoreType.DMA,
        pltpu.SemaphoreType.DMA,
    ]
    if nb > 1:
        scratch += [
            pltpu.VMEM((bk * ch, d), F32),
            pltpu.SemaphoreType.DMA,
            pltpu.SemaphoreType.DMA,
        ]

    @functools.partial(
        pl.kernel,
        out_type=jax.ShapeDtypeStruct((b, d), F32),
        mesh=mesh,
        scratch_types=scratch,
    )
    def k(tab_h, idx_h, out_h, idx_v, rows0, sg0, so0, *rest):
        wid = lax.axis_index("s") * 2 + lax.axis_index("c")
        base = wid * per_w
        pltpu.sync_copy(idx_h.at[wid], idx_v)

        def fire_g(p, rows, sg):
            for i in range(bk):
                pltpu.async_copy(tab_h.at[idx_v.at[p * bk + i]],
                                 rows.at[pl.ds(i * ch, ch)], sg)

        def wait_g(rows, sg):
            for i in range(bk):
                pltpu.make_async_copy(tab_h.at[pl.ds(0, ch)],
                                      rows.at[pl.ds(i * ch, ch)], sg).wait()

        def fire_o(p, rows, so):
            pltpu.async_copy(rows, out_h.at[pl.ds(base + p * bk * ch, bk * ch)], so)

        def wait_o(rows, so):
            pltpu.make_async_copy(rows, out_h.at[pl.ds(base, bk * ch)], so).wait()

        fire_g(0, rows0, sg0)
        if nb > 1:
            rows1, sg1, so1 = rest

            def pair(pr, _):
                p0 = 2 * pr
                fire_g(p0 + 1, rows1, sg1)
                wait_g(rows0, sg0)
                fire_o(p0, rows0, so0)
                wait_g(rows1, sg1)
                fire_o(p0 + 1, rows1, so1)
                wait_o(rows0, so0)
                fire_g(p0 + 2, rows0, sg0)
                wait_o(rows1, so1)
                return 0

            lax.fori_loop(0, (nb - 1) // 2, pair, 0)
        wait_g(rows0, sg0)
        fire_o(nb - 1, rows0, so0)
        wait_o(rows0, so0)

    return k(table, idx3)


def _inter_body(is_last, xj_r, de_r, ce_r, h_r, offs, cf, w1, b1, w2, b2,
                l2w, l2b, lw, lb, *rest):
    dblk = de_r[...]
    cblk = ce_r[...]
    off = offs[...]
    cfv = cf[0, 0]
    xj3 = xj_r[...]
    ea = jnp.concatenate(
        [jnp.exp(cfv * (dblk[:, j:j + 1] - off) ** 2) for j in range(K)],
        axis=0)
    u = _ssp(_bdot(ea, w1[...]) + b1[...])
    W3 = (_bdot(u, w2[...]) + b2[...]).reshape(K, NBI, HC)
    agg = None
    for j in range(K):
        pj = xj3[j] * (W3[j] * cblk[:, j:j + 1])
        agg = pj if agg is None else agg + pj
    o = _ssp(_bdot(agg, l2w[...]) + l2b[...])
    o = _bdot(o, lw[...]) + lb[...]
    hn = h_r[...] + o
    if not is_last:
        l1n, h_o, xh_o = rest
        h_o[...] = hn
        xh_o[...] = _bdot(hn, l1n[...])
    else:
        pw1, pb1, pa1, pw2, pb2, pa2, pw3, pb3, y_o = rest
        y1 = _prelu(_bdot(hn, pw1[...]) + pb1[...], pa1[0, 0])
        y2 = _prelu(_bdot(y1, pw2[...]) + pb2[...], pa2[0, 0])
        y_o[...] = _bdot(y2, pw3[...]) + pb3[0, 0]


def _full(shape):
    return pl.BlockSpec(shape, lambda i: tuple(0 for _ in shape))


def _interact(xj, d_e, c_e, h, offs, cf, w1p, b1, w2, b2, l2w, l2b, lw, lb,
              extra, is_last):
    in_specs = [
        pl.BlockSpec((K, NBI, HC), lambda i: (0, i, 0)),
        pl.BlockSpec((NBI, K), lambda i: (i, 0)),
        pl.BlockSpec((NBI, K), lambda i: (i, 0)),
        pl.BlockSpec((NBI, HC), lambda i: (i, 0)),
        _full((1, NGP)), _full((1, 1)),
        _full((NGP, HC)), _full((1, HC)), _full((HC, HC)), _full((1, HC)),
        _full((HC, HC)), _full((1, HC)), _full((HC, HC)), _full((1, HC)),
    ]
    if not is_last:
        in_specs += [_full((HC, HC))]
        out_specs = [pl.BlockSpec((NBI, HC), lambda i: (i, 0)),
                     pl.BlockSpec((NBI, HC), lambda i: (i, 0))]
        out_shape = [jax.ShapeDtypeStruct((N, HC), F32),
                     jax.ShapeDtypeStruct((N, HC), F32)]
    else:
        in_specs += [_full((HC, HC // 2)), _full((1, HC // 2)), _full((1, 1)),
                     _full((HC // 2, HC // 4)), _full((1, HC // 4)),
                     _full((1, 1)), _full((HC // 4, 1)), _full((1, 1))]
        out_specs = [pl.BlockSpec((NBI, 1), lambda i: (i, 0))]
        out_shape = [jax.ShapeDtypeStruct((N, 1), F32)]
    return pl.pallas_call(
        functools.partial(_inter_body, is_last),
        grid=(N // NBI,),
        in_specs=in_specs,
        out_specs=out_specs,
        out_shape=out_shape,
    )(xj, d_e, c_e, h, offs, cf, w1p, b1, w2, b2, l2w, l2b, lw, lb, *extra)


def kernel(z, pos, batch, emb, mlp_w1, mlp_b1, mlp_w2, mlp_b2, lin1_w,
           lin2_w, lin2_b, lin_w, lin_b, p_w1, p_b1, p_a1, p_w2, p_b2,
           p_a2, p_w3, p_b3):
    pos = pos.astype(F32)
    batch = batch.astype(I32)
    z = z.astype(I32)

    src, vf = _build_edges(pos, batch)
    src3n = src.reshape(NW, 125, 80)
    src3j = src.T.reshape(NW, 125, 80)

    pos_p = jnp.pad(pos, ((0, 0), (0, HC - 3)))
    ps = _sc_gather(pos_p, src3n, 80, 125, HC, 5)
    d_nm, c_nm = _geom(ps, pos_p, vf)

    offs_lin = jnp.linspace(0.0, CUT, NG).astype(F32)
    offs = jnp.pad(offs_lin, (0, NGP - NG), constant_values=1e6)[None]
    cf = (-0.5 / (offs_lin[1] - offs_lin[0]) ** 2).reshape(1, 1)

    table = _table(emb, lin1_w[0])
    zp = jnp.pad(z, (0, NPAD - N)).reshape(NW, 5, 64)
    hx = _sc_gather(table, zp, 64, 5, 2 * HC, 5)
    h = hx[:N, :HC]
    xh = hx[:N, HC:]

    y = None
    for t in range(NI):
        xj = _sc_gather(xh, src3j, 80, 125, HC, 5).reshape(K, N, HC)
        w1p = jnp.pad(mlp_w1[t], ((0, NGP - NG), (0, 0))).astype(BF16)
        args = (xj, d_nm, c_nm, h, offs, cf, w1p, mlp_b1[t][None],
                mlp_w2[t].astype(BF16), mlp_b2[t][None],
                lin2_w[t].astype(BF16), lin2_b[t][None],
                lin_w[t].astype(BF16), lin_b[t][None])
        if t < NI - 1:
            extra = (lin1_w[t + 1].astype(BF16),)
            h, xh = _interact(*args, extra, is_last=False)
        else:
            extra = (p_w1.astype(BF16), p_b1[None], p_a1[None],
                     p_w2.astype(BF16), p_b2[None], p_a2[None],
                     p_w3.astype(BF16), p_b3[None])
            (y,) = _interact(*args, extra, is_last=True)
    return y
